# speedup vs baseline: 1.6406x; 1.6406x over previous
import functools
import jax
import jax.numpy as jnp
from jax.experimental import pallas as pl
from jax.experimental.pallas import tpu as pltpu

BLK = 128
TI = 2048


def _router_kernel(x_ref, gw_ref, logits_ref, meta_ref):
    x = x_ref[...]
    gw = gw_ref[...]
    l = jnp.dot(x, gw, preferred_element_type=jnp.float32)
    logits_ref[...] = l
    lane = jax.lax.broadcasted_iota(jnp.int32, l.shape, 1)
    valid = lane < 8
    neg = jnp.float32(-jnp.inf)
    lm = jnp.where(valid, l, neg)
    m1 = jnp.max(lm, axis=1, keepdims=True)
    i1 = jnp.min(jnp.where(lm == m1, lane, 128), axis=1, keepdims=True)
    s = jnp.sum(jnp.where(valid, jnp.exp(lm - m1), 0.0), axis=1, keepdims=True)
    lm2 = jnp.where(lane == i1, neg, lm)
    m2 = jnp.max(lm2, axis=1, keepdims=True)
    i2 = jnp.min(jnp.where(lm2 == m2, lane, 128), axis=1, keepdims=True)
    w1 = 1.0 / s
    w2 = jnp.exp(m2 - m1) / s
    meta = (jnp.where(lane == 0, i1.astype(jnp.float32), 0.0)
            + jnp.where(lane == 1, i2.astype(jnp.float32), 0.0)
            + jnp.where(lane == 2, w1, 0.0)
            + jnp.where(lane == 3, w2, 0.0))
    meta_ref[...] = meta


def _ffn_kernel(be_ref, xg_ref, win_ref, wv_ref, wout_ref, wt_ref, y_ref,
                acc_ref):
    i = pl.program_id(1)
    ni = pl.num_programs(1)
    xb = xg_ref[...]
    win = win_ref[0]
    wv = wv_ref[0]
    wout = wout_ref[0]
    up = jnp.dot(xb, win, preferred_element_type=jnp.float32)
    v = jnp.dot(xb, wv, preferred_element_type=jnp.float32)
    hg = jax.nn.gelu(up) * v
    part = jnp.dot(hg, wout, preferred_element_type=jnp.float32)

    @pl.when(i == 0)
    def _():
        acc_ref[...] = jnp.zeros_like(acc_ref)

    acc_ref[...] += part

    @pl.when(i == ni - 1)
    def _():
        y_ref[...] = acc_ref[...] * wt_ref[0]


def kernel(hidden, gate_w, w_in, w_v, w_out):
    b, s, h = hidden.shape
    e = gate_w.shape[1]
    ii = w_in.shape[2]
    t = b * s
    x = hidden.reshape(t, h)

    gw_pad = jnp.zeros((h, 128), jnp.float32).at[:, :e].set(gate_w)
    rblk = 256
    logits_pad, meta = pl.pallas_call(
        _router_kernel,
        grid=(t // rblk,),
        in_specs=[
            pl.BlockSpec((rblk, h), lambda i: (i, 0)),
            pl.BlockSpec((h, 128), lambda i: (0, 0)),
        ],
        out_specs=[
            pl.BlockSpec((rblk, 128), lambda i: (i, 0)),
            pl.BlockSpec((rblk, 128), lambda i: (i, 0)),
        ],
        out_shape=[
            jax.ShapeDtypeStruct((t, 128), jnp.float32),
            jax.ShapeDtypeStruct((t, 128), jnp.float32),
        ],
    )(x, gw_pad)
    router_logits = logits_pad[:, :e]
    i1 = meta[:, 0].astype(jnp.int32)
    i2 = meta[:, 1].astype(jnp.int32)
    w1 = meta[:, 2]
    w2 = meta[:, 3]

    cap = 2 * t + e * BLK
    nb = cap // BLK
    ids = jnp.concatenate([i1, i2])
    toks = jnp.concatenate([jnp.arange(t, dtype=jnp.int32)] * 2)
    wts = jnp.concatenate([w1, w2])
    onehot = (ids[:, None] == jnp.arange(e, dtype=jnp.int32)[None, :])
    csum = jnp.cumsum(onehot.astype(jnp.int32), axis=0)
    rank = jnp.sum(csum * onehot, axis=1) - 1
    counts = csum[-1]
    padded = ((counts + BLK - 1) // BLK) * BLK
    offs = jnp.concatenate([jnp.zeros((1,), jnp.int32),
                            jnp.cumsum(padded)[:-1].astype(jnp.int32)])
    pos = offs[ids] + rank
    disp_tok = jnp.zeros((cap,), jnp.int32).at[pos].set(toks)
    disp_wt = jnp.zeros((cap,), jnp.float32).at[pos].set(wts)
    blk_offs = offs // BLK
    block_expert = jnp.sum(
        jnp.arange(nb, dtype=jnp.int32)[:, None] >= blk_offs[None, :],
        axis=1).astype(jnp.int32) - 1
    block_expert = jnp.clip(block_expert, 0, e - 1)
    p1, p2 = pos[:t], pos[t:]

    xg = x[disp_tok]

    ni = ii // TI
    wtb = disp_wt.reshape(nb, BLK, 1)
    grid_spec = pltpu.PrefetchScalarGridSpec(
        num_scalar_prefetch=1,
        grid=(nb, ni),
        in_specs=[
            pl.BlockSpec((BLK, h), lambda bb, i, be: (bb, 0)),
            pl.BlockSpec((1, h, TI), lambda bb, i, be: (be[bb], 0, i)),
            pl.BlockSpec((1, h, TI), lambda bb, i, be: (be[bb], 0, i)),
            pl.BlockSpec((1, TI, h), lambda bb, i, be: (be[bb], i, 0)),
            pl.BlockSpec((1, BLK, 1), lambda bb, i, be: (bb, 0, 0)),
        ],
        out_specs=pl.BlockSpec((BLK, h), lambda bb, i, be: (bb, 0)),
        scratch_shapes=[pltpu.VMEM((BLK, h), jnp.float32)],
    )
    y = pl.pallas_call(
        _ffn_kernel,
        grid_spec=grid_spec,
        out_shape=jax.ShapeDtypeStruct((cap, h), jnp.float32),
        compiler_params=pltpu.CompilerParams(
            dimension_semantics=("arbitrary", "arbitrary")),
    )(block_expert, xg, w_in, w_v, w_out, wtb)

    out = y[p1] + y[p2]
    return out.reshape(b, s, h), router_logits


# speedup vs baseline: 2.4923x; 1.5191x over previous
import functools
import jax
import jax.numpy as jnp
from jax.experimental import pallas as pl
from jax.experimental.pallas import tpu as pltpu

BLK = 128
TI = 1024


def _router_kernel(x_ref, gw_ref, logits_ref, meta_ref):
    x = x_ref[...]
    gw = gw_ref[...]
    l = jnp.dot(x, gw, preferred_element_type=jnp.float32)
    logits_ref[...] = l
    lane = jax.lax.broadcasted_iota(jnp.int32, l.shape, 1)
    valid = lane < 8
    neg = jnp.float32(-jnp.inf)
    lm = jnp.where(valid, l, neg)
    m1 = jnp.max(lm, axis=1, keepdims=True)
    i1 = jnp.min(jnp.where(lm == m1, lane, 128), axis=1, keepdims=True)
    s = jnp.sum(jnp.where(valid, jnp.exp(lm - m1), 0.0), axis=1, keepdims=True)
    lm2 = jnp.where(lane == i1, neg, lm)
    m2 = jnp.max(lm2, axis=1, keepdims=True)
    i2 = jnp.min(jnp.where(lm2 == m2, lane, 128), axis=1, keepdims=True)
    w1 = 1.0 / s
    w2 = jnp.exp(m2 - m1) / s
    meta = (jnp.where(lane == 0, i1.astype(jnp.float32), 0.0)
            + jnp.where(lane == 1, i2.astype(jnp.float32), 0.0)
            + jnp.where(lane == 2, w1, 0.0)
            + jnp.where(lane == 3, w2, 0.0))
    meta_ref[...] = meta


def _ffn_kernel(be_ref, xg_ref, win_ref, wv_ref, wout_ref, wt_ref, y_ref,
                winb_ref, wvb_ref, woutb_ref):
    i = pl.program_id(0)
    bb = pl.program_id(1)
    ni = pl.num_programs(0)
    prev = be_ref[jnp.maximum(bb - 1, 0)]
    changed = jnp.logical_or(bb == 0, be_ref[bb] != prev)

    @pl.when(changed)
    def _():
        winb_ref[...] = win_ref[0].astype(jnp.bfloat16)
        wvb_ref[...] = wv_ref[0].astype(jnp.bfloat16)
        woutb_ref[...] = wout_ref[0].astype(jnp.bfloat16)

    xb = xg_ref[...]
    up = jnp.dot(xb, winb_ref[...], preferred_element_type=jnp.float32)
    v = jnp.dot(xb, wvb_ref[...], preferred_element_type=jnp.float32)
    hg = (jax.nn.gelu(up) * v).astype(jnp.bfloat16)
    part = jnp.dot(hg, woutb_ref[...], preferred_element_type=jnp.float32)

    rows = pl.ds(bb * BLK, BLK)
    if ni == 1:
        y_ref[rows, :] = part * wt_ref[0]
    else:
        @pl.when(i == 0)
        def _():
            y_ref[rows, :] = part

        @pl.when(jnp.logical_and(i > 0, i < ni - 1))
        def _():
            y_ref[rows, :] += part

        @pl.when(i == ni - 1)
        def _():
            y_ref[rows, :] = (y_ref[rows, :] + part) * wt_ref[0]


def kernel(hidden, gate_w, w_in, w_v, w_out):
    b, s, h = hidden.shape
    e = gate_w.shape[1]
    ii = w_in.shape[2]
    t = b * s
    x = hidden.reshape(t, h)

    gw_pad = jnp.zeros((h, 128), jnp.float32).at[:, :e].set(gate_w)
    rblk = 256
    logits_pad, meta = pl.pallas_call(
        _router_kernel,
        grid=(t // rblk,),
        in_specs=[
            pl.BlockSpec((rblk, h), lambda i: (i, 0)),
            pl.BlockSpec((h, 128), lambda i: (0, 0)),
        ],
        out_specs=[
            pl.BlockSpec((rblk, 128), lambda i: (i, 0)),
            pl.BlockSpec((rblk, 128), lambda i: (i, 0)),
        ],
        out_shape=[
            jax.ShapeDtypeStruct((t, 128), jnp.float32),
            jax.ShapeDtypeStruct((t, 128), jnp.float32),
        ],
    )(x, gw_pad)
    router_logits = logits_pad[:, :e]
    i1 = meta[:, 0].astype(jnp.int32)
    i2 = meta[:, 1].astype(jnp.int32)
    w1 = meta[:, 2]
    w2 = meta[:, 3]

    cap = 2 * t + e * BLK
    nb = cap // BLK
    ids = jnp.concatenate([i1, i2])
    toks = jnp.concatenate([jnp.arange(t, dtype=jnp.int32)] * 2)
    wts = jnp.concatenate([w1, w2])
    onehot = (ids[:, None] == jnp.arange(e, dtype=jnp.int32)[None, :])
    csum = jnp.cumsum(onehot.astype(jnp.int32), axis=0)
    rank = jnp.sum(csum * onehot, axis=1) - 1
    counts = csum[-1]
    padded = ((counts + BLK - 1) // BLK) * BLK
    offs = jnp.concatenate([jnp.zeros((1,), jnp.int32),
                            jnp.cumsum(padded)[:-1].astype(jnp.int32)])
    pos = offs[ids] + rank
    disp_tok = jnp.zeros((cap,), jnp.int32).at[pos].set(toks)
    disp_wt = jnp.zeros((cap,), jnp.float32).at[pos].set(wts)
    blk_offs = offs // BLK
    block_expert = jnp.sum(
        jnp.arange(nb, dtype=jnp.int32)[:, None] >= blk_offs[None, :],
        axis=1).astype(jnp.int32) - 1
    block_expert = jnp.clip(block_expert, 0, e - 1)
    p1, p2 = pos[:t], pos[t:]

    xg = x[disp_tok]

    ni = ii // TI
    wtb = disp_wt.reshape(nb, BLK, 1)
    xgb = xg.astype(jnp.bfloat16)
    grid_spec = pltpu.PrefetchScalarGridSpec(
        num_scalar_prefetch=1,
        grid=(ni, nb),
        in_specs=[
            pl.BlockSpec((BLK, h), lambda i, bb, be: (bb, 0)),
            pl.BlockSpec((1, h, TI), lambda i, bb, be: (be[bb], 0, i)),
            pl.BlockSpec((1, h, TI), lambda i, bb, be: (be[bb], 0, i)),
            pl.BlockSpec((1, TI, h), lambda i, bb, be: (be[bb], i, 0)),
            pl.BlockSpec((1, BLK, 1), lambda i, bb, be: (bb, 0, 0)),
        ],
        out_specs=pl.BlockSpec((cap, h), lambda i, bb, be: (0, 0)),
        scratch_shapes=[
            pltpu.VMEM((h, TI), jnp.bfloat16),
            pltpu.VMEM((h, TI), jnp.bfloat16),
            pltpu.VMEM((TI, h), jnp.bfloat16),
        ],
    )
    y = pl.pallas_call(
        _ffn_kernel,
        grid_spec=grid_spec,
        out_shape=jax.ShapeDtypeStruct((cap, h), jnp.float32),
        compiler_params=pltpu.CompilerParams(
            dimension_semantics=("arbitrary", "arbitrary")),
    )(block_expert, xgb, w_in, w_v, w_out, wtb)

    out = y[p1] + y[p2]
    return out.reshape(b, s, h), router_logits


# speedup vs baseline: 3.0027x; 1.2048x over previous
import functools
import jax
import jax.numpy as jnp
from jax.experimental import pallas as pl
from jax.experimental.pallas import tpu as pltpu

BLK = 128
TI = 2048


def _router_kernel(x_ref, gw_ref, logits_ref, meta_ref):
    x = x_ref[...]
    gw = gw_ref[...]
    l = jnp.dot(x, gw, preferred_element_type=jnp.float32)
    logits_ref[...] = l
    lane = jax.lax.broadcasted_iota(jnp.int32, l.shape, 1)
    valid = lane < 8
    neg = jnp.float32(-jnp.inf)
    lm = jnp.where(valid, l, neg)
    m1 = jnp.max(lm, axis=1, keepdims=True)
    i1 = jnp.min(jnp.where(lm == m1, lane, 128), axis=1, keepdims=True)
    s = jnp.sum(jnp.where(valid, jnp.exp(lm - m1), 0.0), axis=1, keepdims=True)
    lm2 = jnp.where(lane == i1, neg, lm)
    m2 = jnp.max(lm2, axis=1, keepdims=True)
    i2 = jnp.min(jnp.where(lm2 == m2, lane, 128), axis=1, keepdims=True)
    w1 = 1.0 / s
    w2 = jnp.exp(m2 - m1) / s
    meta = (jnp.where(lane == 0, i1.astype(jnp.float32), 0.0)
            + jnp.where(lane == 1, i2.astype(jnp.float32), 0.0)
            + jnp.where(lane == 2, w1, 0.0)
            + jnp.where(lane == 3, w2, 0.0))
    meta_ref[...] = meta


def _ffn_kernel(be_ref, xg_ref, win_ref, wv_ref, wout_ref, wt_ref, y_ref,
                winb_ref, wvb_ref, woutb_ref):
    i = pl.program_id(0)
    bb = pl.program_id(1)
    ni = pl.num_programs(0)
    prev = be_ref[jnp.maximum(bb - 1, 0)]
    changed = jnp.logical_or(bb == 0, be_ref[bb] != prev)

    @pl.when(changed)
    def _():
        winb_ref[...] = win_ref[0].astype(jnp.bfloat16)
        wvb_ref[...] = wv_ref[0].astype(jnp.bfloat16)
        woutb_ref[...] = wout_ref[0].astype(jnp.bfloat16)

    xb = xg_ref[...]
    up = jnp.dot(xb, winb_ref[...], preferred_element_type=jnp.float32)
    v = jnp.dot(xb, wvb_ref[...], preferred_element_type=jnp.float32)
    hg = (jax.nn.gelu(up) * v).astype(jnp.bfloat16)
    part = jnp.dot(hg, woutb_ref[...], preferred_element_type=jnp.float32)

    rows = pl.ds(bb * BLK, BLK)
    if ni == 1:
        y_ref[rows, :] = part * wt_ref[0]
    else:
        @pl.when(i == 0)
        def _():
            y_ref[rows, :] = part

        @pl.when(jnp.logical_and(i > 0, i < ni - 1))
        def _():
            y_ref[rows, :] += part

        @pl.when(i == ni - 1)
        def _():
            y_ref[rows, :] = (y_ref[rows, :] + part) * wt_ref[0]


def kernel(hidden, gate_w, w_in, w_v, w_out):
    b, s, h = hidden.shape
    e = gate_w.shape[1]
    ii = w_in.shape[2]
    t = b * s
    x = hidden.reshape(t, h)

    gw_pad = jnp.zeros((h, 128), jnp.float32).at[:, :e].set(gate_w)
    rblk = 256
    logits_pad, meta = pl.pallas_call(
        _router_kernel,
        grid=(t // rblk,),
        in_specs=[
            pl.BlockSpec((rblk, h), lambda i: (i, 0)),
            pl.BlockSpec((h, 128), lambda i: (0, 0)),
        ],
        out_specs=[
            pl.BlockSpec((rblk, 128), lambda i: (i, 0)),
            pl.BlockSpec((rblk, 128), lambda i: (i, 0)),
        ],
        out_shape=[
            jax.ShapeDtypeStruct((t, 128), jnp.float32),
            jax.ShapeDtypeStruct((t, 128), jnp.float32),
        ],
    )(x, gw_pad)
    router_logits = logits_pad[:, :e]
    i1 = meta[:, 0].astype(jnp.int32)
    i2 = meta[:, 1].astype(jnp.int32)
    w1 = meta[:, 2]
    w2 = meta[:, 3]

    cap = 2 * t + e * BLK
    nb = cap // BLK
    ids = jnp.concatenate([i1, i2])
    toks = jnp.concatenate([jnp.arange(t, dtype=jnp.int32)] * 2)
    wts = jnp.concatenate([w1, w2])
    onehot = (ids[:, None] == jnp.arange(e, dtype=jnp.int32)[None, :])
    csum = jnp.cumsum(onehot.astype(jnp.int32), axis=0)
    rank = jnp.sum(csum * onehot, axis=1) - 1
    counts = csum[-1]
    padded = ((counts + BLK - 1) // BLK) * BLK
    offs = jnp.concatenate([jnp.zeros((1,), jnp.int32),
                            jnp.cumsum(padded)[:-1].astype(jnp.int32)])
    pos = offs[ids] + rank
    disp_tok = jnp.zeros((cap,), jnp.int32).at[pos].set(toks)
    disp_wt = jnp.zeros((cap,), jnp.float32).at[pos].set(wts)
    blk_offs = offs // BLK
    block_expert = jnp.sum(
        jnp.arange(nb, dtype=jnp.int32)[:, None] >= blk_offs[None, :],
        axis=1).astype(jnp.int32) - 1
    block_expert = jnp.clip(block_expert, 0, e - 1)
    p1, p2 = pos[:t], pos[t:]

    xg = x[disp_tok]

    ni = ii // TI
    wtb = disp_wt.reshape(nb, BLK, 1)
    xgb = xg.astype(jnp.bfloat16)
    grid_spec = pltpu.PrefetchScalarGridSpec(
        num_scalar_prefetch=1,
        grid=(ni, nb),
        in_specs=[
            pl.BlockSpec((BLK, h), lambda i, bb, be: (bb, 0)),
            pl.BlockSpec((1, h, TI), lambda i, bb, be: (be[bb], 0, i)),
            pl.BlockSpec((1, h, TI), lambda i, bb, be: (be[bb], 0, i)),
            pl.BlockSpec((1, TI, h), lambda i, bb, be: (be[bb], i, 0)),
            pl.BlockSpec((1, BLK, 1), lambda i, bb, be: (bb, 0, 0)),
        ],
        out_specs=pl.BlockSpec((cap, h), lambda i, bb, be: (0, 0)),
        scratch_shapes=[
            pltpu.VMEM((h, TI), jnp.bfloat16),
            pltpu.VMEM((h, TI), jnp.bfloat16),
            pltpu.VMEM((TI, h), jnp.bfloat16),
        ],
    )
    y = pl.pallas_call(
        _ffn_kernel,
        grid_spec=grid_spec,
        out_shape=jax.ShapeDtypeStruct((cap, h), jnp.float32),
        compiler_params=pltpu.CompilerParams(
            dimension_semantics=("arbitrary", "arbitrary"),
            vmem_limit_bytes=120 * 1024 * 1024),
    )(block_expert, xgb, w_in, w_v, w_out, wtb)

    out = y[p1] + y[p2]
    return out.reshape(b, s, h), router_logits
